# baseline (speedup 1.0000x reference)
import functools

import jax
import jax.numpy as jnp
from jax import lax
from jax.experimental import pallas as pl
from jax.experimental.pallas import tpu as pltpu
from jax.experimental.pallas import tpu_sc as plsc

NC = 2
NS = 16
NW = NC * NS
CHUNK = 128
NBUF = 8
QROWS = 32
Q = QROWS * CHUNK

D_HID = 16
D_OUT = 5


def _mesh():
    return plsc.VectorSubcoreMesh(
        core_axis_name="c", subcore_axis_name="s", num_cores=NC, num_subcores=NS
    )


def _sc_params():
    return pltpu.CompilerParams(needs_layout_passes=False,
                                use_tc_tiling_on_sc=False)


def _partition_call(src2d, dst2d, ones, n_pad, cpw):
    half = n_pad // 2
    regrows = cpw + QROWS
    tot_rows = NW * 2 * regrows
    deg_rows_per_tile = n_pad // NS

    @functools.partial(
        pl.kernel,
        out_type=(
            jax.ShapeDtypeStruct((NC * n_pad,), jnp.float32),
            jax.ShapeDtypeStruct((tot_rows, CHUNK), jnp.int32),
            jax.ShapeDtypeStruct((tot_rows, CHUNK), jnp.int32),
            jax.ShapeDtypeStruct((NW * 2 * 16,), jnp.int32),
        ),
        mesh=_mesh(),
        scratch_types=[
            pltpu.VMEM((2, NBUF, CHUNK), jnp.int32),
            pltpu.VMEM((2, NBUF, CHUNK), jnp.int32),
            pltpu.VMEM((CHUNK,), jnp.float32),
            pltpu.VMEM((QROWS + 1, CHUNK), jnp.int32),
            pltpu.VMEM((QROWS + 1, CHUNK), jnp.int32),
            pltpu.VMEM((QROWS + 1, CHUNK), jnp.int32),
            pltpu.VMEM((QROWS + 1, CHUNK), jnp.int32),
            pltpu.VMEM((deg_rows_per_tile,), jnp.float32),
            pltpu.VMEM((16,), jnp.int32),
            pltpu.VMEM_SHARED((n_pad,), jnp.float32),
            pltpu.SemaphoreType.DMA,
            pltpu.SemaphoreType.DMA,
        ],
        compiler_params=_sc_params(),
    )
    def part_kernel(src_hbm, dst_hbm, ones_hbm, deg_hbm, psrc_hbm, pdst_hbm,
                    cnt_hbm, sstage, dstage, ones_v, bs0, bd0, bs1, bd1,
                    bounce_v, cvec_v, deg_sh, sem_i, sem_d):
        cid = lax.axis_index("c")
        sid = lax.axis_index("s")
        wid = sid * NC + cid
        base_n = sid * deg_rows_per_tile
        pltpu.sync_copy(ones_hbm.at[pl.ds(base_n, deg_rows_per_tile)], bounce_v)
        pltpu.sync_copy(bounce_v, deg_sh.at[pl.ds(base_n, deg_rows_per_tile)])
        pltpu.sync_copy(ones_hbm.at[pl.ds(0, CHUNK)], ones_v)
        plsc.subcore_barrier()

        row0 = wid * cpw
        bufs = ((bs0, bd0), (bs1, bd1))
        rb0 = (wid * 2) * regrows
        rb1 = (wid * 2 + 1) * regrows
        n_it = cpw // NBUF
        iota1 = lax.iota(jnp.int32, 16) + 1

        pltpu.async_copy(src_hbm.at[pl.ds(row0, NBUF)], sstage.at[0], sem_i)
        pltpu.async_copy(dst_hbm.at[pl.ds(row0, NBUF)], dstage.at[0], sem_i)

        def body(i, carry):
            f0, w0, f1, w1 = carry
            b = i % 2
            pltpu.make_async_copy(src_hbm.at[pl.ds(0, NBUF)], sstage.at[b],
                                  sem_i).wait()
            pltpu.make_async_copy(src_hbm.at[pl.ds(0, NBUF)], dstage.at[b],
                                  sem_i).wait()
            nxt = row0 + jnp.minimum(i + 1, n_it - 1) * NBUF
            pltpu.async_copy(src_hbm.at[pl.ds(nxt, NBUF)], sstage.at[1 - b],
                             sem_i)
            pltpu.async_copy(dst_hbm.at[pl.ds(nxt, NBUF)], dstage.at[1 - b],
                             sem_i)
            dcps = [pltpu.async_copy(ones_v, deg_sh.at[dstage.at[b, j]],
                                     sem_d, add=True)
                    for j in range(NBUF)]
            fills = [jnp.zeros((16,), jnp.int32) + f0,
                     jnp.zeros((16,), jnp.int32) + f1]
            wrows = [w0, w1]
            for j in range(NBUF):
                for k in range(CHUNK // 16):
                    s = sstage[b, j, pl.ds(k * 16, 16)]
                    d = dstage[b, j, pl.ds(k * 16, 16)]
                    m0 = d < half
                    ones_i = jnp.where(m0, 1, 0).astype(jnp.int32)
                    csum0 = plsc.cumsum(ones_i)
                    cnt0 = plsc.all_reduce_population_count(m0)
                    pos = fills[0] + csum0 - 1
                    prow = lax.shift_right_logical(pos, 7)
                    pcol = jnp.bitwise_and(pos, 127)
                    plsc.store_scatter(bs0, [prow, pcol], s, mask=m0)
                    plsc.store_scatter(bd0, [prow, pcol], d, mask=m0)
                    fills[0] = fills[0] + cnt0
                    m1 = ~m0
                    pos = fills[1] + (iota1 - csum0) - 1
                    prow = lax.shift_right_logical(pos, 7)
                    pcol = jnp.bitwise_and(pos, 127)
                    plsc.store_scatter(bs1, [prow, pcol], s, mask=m1)
                    plsc.store_scatter(bd1, [prow, pcol], d - half, mask=m1)
                    fills[1] = fills[1] + (16 - cnt0)
                for c, rb in ((0, rb0), (1, rb1)):
                    bs, bd = bufs[c]
                    fl = jnp.max(fills[c])
                    wr = wrows[c]

                    @pl.when(fl >= Q)
                    def _(bs=bs, bd=bd, rb=rb, wr=wr):
                        pltpu.sync_copy(bs.at[pl.ds(0, QROWS)],
                                        psrc_hbm.at[pl.ds(rb + wr, QROWS)])
                        pltpu.sync_copy(bd.at[pl.ds(0, QROWS)],
                                        pdst_hbm.at[pl.ds(rb + wr, QROWS)])
                        for k in range(CHUNK // 16):
                            bs[0, pl.ds(k * 16, 16)] = bs[QROWS, pl.ds(k * 16, 16)]
                            bd[0, pl.ds(k * 16, 16)] = bd[QROWS, pl.ds(k * 16, 16)]

                    wrows[c] = jnp.where(fl >= Q, wr + QROWS, wr)
                    fills[c] = jnp.where(fl >= Q, fills[c] - Q, fills[c])
            for cp in dcps:
                cp.wait()
            return (jnp.max(fills[0]), wrows[0], jnp.max(fills[1]), wrows[1])

        z = jnp.int32(0)
        f0, w0, f1, w1 = lax.fori_loop(0, n_it, body, (z, z, z, z))
        pltpu.make_async_copy(src_hbm.at[pl.ds(0, NBUF)], sstage.at[0],
                              sem_i).wait()
        pltpu.make_async_copy(src_hbm.at[pl.ds(0, NBUF)], dstage.at[0],
                              sem_i).wait()

        dummy_s = jnp.zeros((16,), jnp.int32)
        for c, rb, fl, wr in ((0, rb0, f0, w0), (1, rb1, f1, w1)):
            bs, bd = bufs[c]
            dummy_d = jnp.full((16,), half, jnp.int32)
            iota16 = lax.iota(jnp.int32, 16)

            def pad_body(i, carry, bs=bs, bd=bd, fl=fl):
                lanes = i * 16 + iota16

                @pl.when(i * 16 + 16 > fl)
                def _():
                    lrow = lax.shift_right_logical(lanes, 7)
                    lcol = jnp.bitwise_and(lanes, 127)
                    mm = lanes >= fl
                    plsc.store_scatter(bs, [lrow, lcol], dummy_s, mask=mm)
                    plsc.store_scatter(bd, [lrow, lcol], dummy_d, mask=mm)
                return carry

            lax.fori_loop(0, Q // 16, pad_body, 0)

            @pl.when(fl > 0)
            def _(bs=bs, bd=bd, rb=rb, wr=wr):
                pltpu.sync_copy(bs.at[pl.ds(0, QROWS)],
                                psrc_hbm.at[pl.ds(rb + wr, QROWS)])
                pltpu.sync_copy(bd.at[pl.ds(0, QROWS)],
                                pdst_hbm.at[pl.ds(rb + wr, QROWS)])

            wfin = jnp.where(fl > 0, wr + QROWS, wr)
            cvec_v[...] = jnp.zeros((16,), jnp.int32) + wfin
            slot = (wid * 2 + c) * 16
            pltpu.sync_copy(cvec_v,
                            cnt_hbm.at[pl.ds(pl.multiple_of(slot, 8), 16)])

        plsc.subcore_barrier()
        pltpu.sync_copy(deg_sh.at[pl.ds(base_n, deg_rows_per_tile)], bounce_v)
        pltpu.sync_copy(bounce_v,
                        deg_hbm.at[pl.ds(cid * n_pad + base_n,
                                         deg_rows_per_tile)])

    deg, psrc, pdst, cnts = part_kernel(src2d, dst2d, ones)
    return deg.reshape(NC, n_pad), psrc, pdst, cnts


def _edge_scatter_call(psrc, pdst, cnts, g, n_pad, cpw):
    half = n_pad // 2
    rows_per_tile = half // NS
    regrows = cpw + QROWS
    NVOL = 16

    @functools.partial(
        pl.kernel,
        out_type=jax.ShapeDtypeStruct((n_pad, D_HID), jnp.float32),
        mesh=_mesh(),
        scratch_types=[
            pltpu.VMEM((NVOL, CHUNK), jnp.int32),
            pltpu.VMEM((NVOL, CHUNK), jnp.int32),
            pltpu.VMEM((NVOL, CHUNK, D_HID), jnp.float32),
            pltpu.VMEM((rows_per_tile // 8, D_HID), jnp.float32),
            pltpu.VMEM((16,), jnp.int32),
            pltpu.VMEM_SHARED((half + 8, D_HID), jnp.float32),
            pltpu.SemaphoreType.DMA,
            pltpu.SemaphoreType.DMA,
        ],
        compiler_params=_sc_params(),
    )
    def edge_kernel(psrc_hbm, pdst_hbm, cnt_hbm, g_hbm, out_hbm, sidx, didx,
                    rows, bounce_v, cvec_v, acc_sh, sem_g, sem_s):
        cid = lax.axis_index("c")
        sid = lax.axis_index("s")
        lo = cid * half
        base_n = sid * rows_per_tile
        qr = rows_per_tile // 8
        for q in range(8):
            pltpu.sync_copy(g_hbm.at[pl.ds(lo + base_n + q * qr, qr)], bounce_v)
            pltpu.sync_copy(bounce_v, acc_sh.at[pl.ds(base_n + q * qr, qr)])
        plsc.subcore_barrier()

        for p in range(2):
            reg = (2 * sid + p) * 2 + cid
            rowbase = reg * regrows
            slot = reg * 16
            pltpu.sync_copy(cnt_hbm.at[pl.ds(pl.multiple_of(slot, 8), 16)],
                            cvec_v)
            nvol = jnp.max(cvec_v[...]) // NVOL

            def body(v, carry):
                r0 = rowbase + v * NVOL
                icps = [pltpu.async_copy(psrc_hbm.at[pl.ds(r0, NVOL)], sidx,
                                         sem_g),
                        pltpu.async_copy(pdst_hbm.at[pl.ds(r0, NVOL)], didx,
                                         sem_g)]
                for cp in icps:
                    cp.wait()
                gcps = [pltpu.async_copy(g_hbm.at[sidx.at[j]], rows.at[j],
                                         sem_g)
                        for j in range(NVOL)]
                scps = []
                for j in range(NVOL):
                    gcps[j].wait()
                    scps.append(pltpu.async_copy(
                        rows.at[j], acc_sh.at[didx.at[j]], sem_s, add=True))
                for cp in scps:
                    cp.wait()
                return carry

            lax.fori_loop(0, nvol, body, 0)

        plsc.subcore_barrier()
        for q in range(8):
            pltpu.sync_copy(acc_sh.at[pl.ds(base_n + q * qr, qr)], bounce_v)
            pltpu.sync_copy(bounce_v,
                            out_hbm.at[pl.ds(lo + base_n + q * qr, qr)])

    return edge_kernel(psrc, pdst, cnts, g)


def _dense1_body(x_ref, degt_ref, w_ref, g_ref):
    deg = degt_ref[:, 0:1] + degt_ref[:, 1:2] + 1.0
    dis = lax.rsqrt(deg)
    h = jnp.dot(x_ref[...], w_ref[...], preferred_element_type=jnp.float32)
    g_ref[...] = h * dis


def _dense2_body(acc_ref, degt_ref, bc_ref, wf_ref, bf_ref, out_ref):
    deg = degt_ref[:, 0:1] + degt_ref[:, 1:2] + 1.0
    dis = lax.rsqrt(deg)
    t = acc_ref[...] * dis + bc_ref[...]
    t = jnp.maximum(t, 0.0)
    o = jnp.dot(t, wf_ref[...], preferred_element_type=jnp.float32) + bf_ref[...]
    m = jnp.max(o, axis=1, keepdims=True)
    e = jnp.exp(o - m)
    s = jnp.sum(e, axis=1, keepdims=True)
    out_ref[...] = o - m - jnp.log(s)


def kernel(x, edge_index, W_conv, b_conv, W_fc, b_fc):
    n = x.shape[0]
    e = edge_index.shape[1]
    d_in = x.shape[1]

    n_pad = ((n + 1 + NS * 8 - 1) // (NS * 8)) * (NS * 8)
    cpw = -(-e // (NW * CHUNK))
    cpw = ((cpw + 15) // 16) * 16

    src = edge_index[0].astype(jnp.int32)
    dst = edge_index[1].astype(jnp.int32)
    e_pad = NW * CHUNK * cpw
    fill = jnp.full((e_pad - e,), n, jnp.int32)
    src2d = jnp.concatenate([src, fill]).reshape(e_pad // CHUNK, CHUNK)
    dst2d = jnp.concatenate([dst, fill]).reshape(e_pad // CHUNK, CHUNK)
    ones = jnp.ones((n_pad,), jnp.float32)

    degp, psrc, pdst, cnts = _partition_call(src2d, dst2d, ones, n_pad, cpw)
    degt = jnp.swapaxes(degp, 0, 1)

    blk = 8192
    grid = -(-n // blk)

    g = pl.pallas_call(
        _dense1_body,
        grid=(grid,),
        in_specs=[
            pl.BlockSpec((blk, d_in), lambda i: (i, 0)),
            pl.BlockSpec((blk, 2), lambda i: (i, 0)),
            pl.BlockSpec((d_in, D_HID), lambda i: (0, 0)),
        ],
        out_specs=pl.BlockSpec((blk, D_HID), lambda i: (i, 0)),
        out_shape=jax.ShapeDtypeStruct((n_pad, D_HID), jnp.float32),
    )(x, degt, W_conv)

    acc = _edge_scatter_call(psrc, pdst, cnts, g, n_pad, cpw)

    out = pl.pallas_call(
        _dense2_body,
        grid=(grid,),
        in_specs=[
            pl.BlockSpec((blk, D_HID), lambda i: (i, 0)),
            pl.BlockSpec((blk, 2), lambda i: (i, 0)),
            pl.BlockSpec((1, D_HID), lambda i: (0, 0)),
            pl.BlockSpec((D_HID, D_OUT), lambda i: (0, 0)),
            pl.BlockSpec((1, D_OUT), lambda i: (0, 0)),
        ],
        out_specs=pl.BlockSpec((blk, D_OUT), lambda i: (i, 0)),
        out_shape=jax.ShapeDtypeStruct((n, D_OUT), jnp.float32),
    )(acc, degt, b_conv.reshape(1, D_HID), W_fc, b_fc.reshape(1, D_OUT))

    return out


# speedup vs baseline: 2.1604x; 2.1604x over previous
import functools

import jax
import jax.numpy as jnp
from jax import lax
from jax.experimental import pallas as pl
from jax.experimental.pallas import tpu as pltpu
from jax.experimental.pallas import tpu_sc as plsc

NC = 2
NS = 16
NW = NC * NS
CHUNK = 128
NBUF = 8
QROWS = 32
Q = QROWS * CHUNK

D_HID = 16
D_OUT = 5


def _mesh():
    return plsc.VectorSubcoreMesh(
        core_axis_name="c", subcore_axis_name="s", num_cores=NC, num_subcores=NS
    )


def _sc_params():
    return pltpu.CompilerParams(needs_layout_passes=False,
                                use_tc_tiling_on_sc=False)


def _partition_call(src2d, dst2d, ones, n_pad, cpw):
    half = n_pad // 2
    regrows = cpw + QROWS
    tot_rows = NW * 2 * regrows
    deg_rows_per_tile = n_pad // NS

    @functools.partial(
        pl.kernel,
        out_type=(
            jax.ShapeDtypeStruct((NC * n_pad,), jnp.float32),
            jax.ShapeDtypeStruct((tot_rows, CHUNK), jnp.int32),
            jax.ShapeDtypeStruct((tot_rows, CHUNK), jnp.int32),
            jax.ShapeDtypeStruct((NW * 2 * 16,), jnp.int32),
        ),
        mesh=_mesh(),
        scratch_types=[
            pltpu.VMEM((2, NBUF, CHUNK), jnp.int32),
            pltpu.VMEM((2, NBUF, CHUNK), jnp.int32),
            pltpu.VMEM((CHUNK,), jnp.float32),
            pltpu.VMEM((QROWS + 1, CHUNK), jnp.int32),
            pltpu.VMEM((QROWS + 1, CHUNK), jnp.int32),
            pltpu.VMEM((QROWS + 1, CHUNK), jnp.int32),
            pltpu.VMEM((QROWS + 1, CHUNK), jnp.int32),
            pltpu.VMEM((deg_rows_per_tile,), jnp.float32),
            pltpu.VMEM((16,), jnp.int32),
            pltpu.VMEM_SHARED((n_pad,), jnp.float32),
            pltpu.SemaphoreType.DMA,
            pltpu.SemaphoreType.DMA,
        ],
        compiler_params=_sc_params(),
    )
    def part_kernel(src_hbm, dst_hbm, ones_hbm, deg_hbm, psrc_hbm, pdst_hbm,
                    cnt_hbm, sstage, dstage, ones_v, bs0, bd0, bs1, bd1,
                    bounce_v, cvec_v, deg_sh, sem_i, sem_d):
        cid = lax.axis_index("c")
        sid = lax.axis_index("s")
        wid = sid * NC + cid
        base_n = sid * deg_rows_per_tile
        pltpu.sync_copy(ones_hbm.at[pl.ds(base_n, deg_rows_per_tile)], bounce_v)
        pltpu.sync_copy(bounce_v, deg_sh.at[pl.ds(base_n, deg_rows_per_tile)])
        pltpu.sync_copy(ones_hbm.at[pl.ds(0, CHUNK)], ones_v)
        plsc.subcore_barrier()

        row0 = wid * cpw
        bufs = ((bs0, bd0), (bs1, bd1))
        rb0 = (wid * 2) * regrows
        rb1 = (wid * 2 + 1) * regrows
        n_it = cpw // NBUF
        iota1 = lax.iota(jnp.int32, 16) + 1

        pltpu.async_copy(src_hbm.at[pl.ds(row0, NBUF)], sstage.at[0], sem_i)
        pltpu.async_copy(dst_hbm.at[pl.ds(row0, NBUF)], dstage.at[0], sem_i)

        def body(i, carry):
            f0, w0, f1, w1 = carry
            b = i % 2
            pltpu.make_async_copy(src_hbm.at[pl.ds(0, NBUF)], sstage.at[b],
                                  sem_i).wait()
            pltpu.make_async_copy(src_hbm.at[pl.ds(0, NBUF)], dstage.at[b],
                                  sem_i).wait()
            nxt = row0 + jnp.minimum(i + 1, n_it - 1) * NBUF
            pltpu.async_copy(src_hbm.at[pl.ds(nxt, NBUF)], sstage.at[1 - b],
                             sem_i)
            pltpu.async_copy(dst_hbm.at[pl.ds(nxt, NBUF)], dstage.at[1 - b],
                             sem_i)
            dcps = [pltpu.async_copy(ones_v, deg_sh.at[dstage.at[b, j]],
                                     sem_d, add=True)
                    for j in range(NBUF)]
            fills = [jnp.zeros((16,), jnp.int32) + f0,
                     jnp.zeros((16,), jnp.int32) + f1]
            wrows = [w0, w1]
            for j in range(NBUF):
                for k in range(CHUNK // 16):
                    s = sstage[b, j, pl.ds(k * 16, 16)]
                    d = dstage[b, j, pl.ds(k * 16, 16)]
                    m0 = d < half
                    ones_i = jnp.where(m0, 1, 0).astype(jnp.int32)
                    csum0 = plsc.cumsum(ones_i)
                    cnt0 = plsc.all_reduce_population_count(m0)
                    pos = fills[0] + csum0 - 1
                    prow = lax.shift_right_logical(pos, 7)
                    pcol = jnp.bitwise_and(pos, 127)
                    plsc.store_scatter(bs0, [prow, pcol], s, mask=m0)
                    plsc.store_scatter(bd0, [prow, pcol], d, mask=m0)
                    fills[0] = fills[0] + cnt0
                    m1 = ~m0
                    pos = fills[1] + (iota1 - csum0) - 1
                    prow = lax.shift_right_logical(pos, 7)
                    pcol = jnp.bitwise_and(pos, 127)
                    plsc.store_scatter(bs1, [prow, pcol], s, mask=m1)
                    plsc.store_scatter(bd1, [prow, pcol], d - half, mask=m1)
                    fills[1] = fills[1] + (16 - cnt0)
                for c, rb in ((0, rb0), (1, rb1)):
                    bs, bd = bufs[c]
                    fl = jnp.max(fills[c])
                    wr = wrows[c]

                    @pl.when(fl >= Q)
                    def _(bs=bs, bd=bd, rb=rb, wr=wr):
                        pltpu.sync_copy(bs.at[pl.ds(0, QROWS)],
                                        psrc_hbm.at[pl.ds(rb + wr, QROWS)])
                        pltpu.sync_copy(bd.at[pl.ds(0, QROWS)],
                                        pdst_hbm.at[pl.ds(rb + wr, QROWS)])
                        for k in range(CHUNK // 16):
                            bs[0, pl.ds(k * 16, 16)] = bs[QROWS, pl.ds(k * 16, 16)]
                            bd[0, pl.ds(k * 16, 16)] = bd[QROWS, pl.ds(k * 16, 16)]

                    wrows[c] = jnp.where(fl >= Q, wr + QROWS, wr)
                    fills[c] = jnp.where(fl >= Q, fills[c] - Q, fills[c])
            for cp in dcps:
                cp.wait()
            return (jnp.max(fills[0]), wrows[0], jnp.max(fills[1]), wrows[1])

        z = jnp.int32(0)
        f0, w0, f1, w1 = lax.fori_loop(0, n_it, body, (z, z, z, z))
        pltpu.make_async_copy(src_hbm.at[pl.ds(0, NBUF)], sstage.at[0],
                              sem_i).wait()
        pltpu.make_async_copy(src_hbm.at[pl.ds(0, NBUF)], dstage.at[0],
                              sem_i).wait()

        for c, rb, fl, wr in ((0, rb0, f0, w0), (1, rb1, f1, w1)):
            bs, bd = bufs[c]
            iota16 = lax.iota(jnp.int32, 16)

            def pad_body(i, carry, bs=bs, bd=bd, fl=fl):
                lanes = i * 16 + iota16

                @pl.when(i * 16 + 16 > fl)
                def _():
                    lrow = lax.shift_right_logical(lanes, 7)
                    lcol = jnp.bitwise_and(lanes, 127)
                    mm = lanes >= fl
                    plsc.store_scatter(bs, [lrow, lcol], lcol, mask=mm)
                    plsc.store_scatter(bd, [lrow, lcol], half + lcol, mask=mm)
                return carry

            lax.fori_loop(0, Q // 16, pad_body, 0)

            @pl.when(fl > 0)
            def _(bs=bs, bd=bd, rb=rb, wr=wr):
                pltpu.sync_copy(bs.at[pl.ds(0, QROWS)],
                                psrc_hbm.at[pl.ds(rb + wr, QROWS)])
                pltpu.sync_copy(bd.at[pl.ds(0, QROWS)],
                                pdst_hbm.at[pl.ds(rb + wr, QROWS)])

            wfin = jnp.where(fl > 0, wr + QROWS, wr)
            cvec_v[...] = jnp.zeros((16,), jnp.int32) + wfin
            slot = (wid * 2 + c) * 16
            pltpu.sync_copy(cvec_v,
                            cnt_hbm.at[pl.ds(pl.multiple_of(slot, 8), 16)])

        plsc.subcore_barrier()
        pltpu.sync_copy(deg_sh.at[pl.ds(base_n, deg_rows_per_tile)], bounce_v)
        pltpu.sync_copy(bounce_v,
                        deg_hbm.at[pl.ds(cid * n_pad + base_n,
                                         deg_rows_per_tile)])

    deg, psrc, pdst, cnts = part_kernel(src2d, dst2d, ones)
    return deg.reshape(NC, n_pad), psrc, pdst, cnts


def _edge_scatter_call(psrc, pdst, cnts, g, n_pad, cpw):
    half = n_pad // 2
    rows_per_tile = half // NS
    regrows = cpw + QROWS
    NVOL = 16

    @functools.partial(
        pl.kernel,
        out_type=jax.ShapeDtypeStruct((n_pad, D_HID), jnp.float32),
        mesh=_mesh(),
        scratch_types=[
            pltpu.VMEM((NVOL, CHUNK), jnp.int32),
            pltpu.VMEM((NVOL, CHUNK), jnp.int32),
            pltpu.VMEM((NVOL, CHUNK, D_HID), jnp.float32),
            pltpu.VMEM((rows_per_tile // 8, D_HID), jnp.float32),
            pltpu.VMEM((16,), jnp.int32),
            pltpu.VMEM_SHARED((half + CHUNK, D_HID), jnp.float32),
            pltpu.SemaphoreType.DMA,
            pltpu.SemaphoreType.DMA,
        ],
        compiler_params=_sc_params(),
    )
    def edge_kernel(psrc_hbm, pdst_hbm, cnt_hbm, g_hbm, out_hbm, sidx, didx,
                    rows, bounce_v, cvec_v, acc_sh, sem_g, sem_s):
        cid = lax.axis_index("c")
        sid = lax.axis_index("s")
        lo = cid * half
        base_n = sid * rows_per_tile
        qr = rows_per_tile // 8
        for q in range(8):
            pltpu.sync_copy(g_hbm.at[pl.ds(lo + base_n + q * qr, qr)], bounce_v)
            pltpu.sync_copy(bounce_v, acc_sh.at[pl.ds(base_n + q * qr, qr)])
        plsc.subcore_barrier()

        for p in range(2):
            reg = (2 * sid + p) * 2 + cid
            rowbase = reg * regrows
            slot = reg * 16
            pltpu.sync_copy(cnt_hbm.at[pl.ds(pl.multiple_of(slot, 8), 16)],
                            cvec_v)
            nvol = jnp.max(cvec_v[...]) // NVOL

            def body(v, carry):
                r0 = rowbase + v * NVOL
                icps = [pltpu.async_copy(psrc_hbm.at[pl.ds(r0, NVOL)], sidx,
                                         sem_g),
                        pltpu.async_copy(pdst_hbm.at[pl.ds(r0, NVOL)], didx,
                                         sem_g)]
                for cp in icps:
                    cp.wait()
                gcps = [pltpu.async_copy(g_hbm.at[sidx.at[j]], rows.at[j],
                                         sem_g)
                        for j in range(NVOL)]
                scps = []
                for j in range(NVOL):
                    gcps[j].wait()
                    scps.append(pltpu.async_copy(
                        rows.at[j], acc_sh.at[didx.at[j]], sem_s, add=True))
                for cp in scps:
                    cp.wait()
                return carry

            lax.fori_loop(0, nvol, body, 0)

        plsc.subcore_barrier()
        for q in range(8):
            pltpu.sync_copy(acc_sh.at[pl.ds(base_n + q * qr, qr)], bounce_v)
            pltpu.sync_copy(bounce_v,
                            out_hbm.at[pl.ds(lo + base_n + q * qr, qr)])

    return edge_kernel(psrc, pdst, cnts, g)


def _dense1_body(x_ref, degt_ref, w_ref, g_ref):
    deg = degt_ref[:, 0:1] + degt_ref[:, 1:2] + 1.0
    dis = lax.rsqrt(deg)
    h = jnp.dot(x_ref[...], w_ref[...], preferred_element_type=jnp.float32)
    g_ref[...] = h * dis


def _dense2_body(acc_ref, degt_ref, bc_ref, wf_ref, bf_ref, out_ref):
    deg = degt_ref[:, 0:1] + degt_ref[:, 1:2] + 1.0
    dis = lax.rsqrt(deg)
    t = acc_ref[...] * dis + bc_ref[...]
    t = jnp.maximum(t, 0.0)
    o = jnp.dot(t, wf_ref[...], preferred_element_type=jnp.float32) + bf_ref[...]
    m = jnp.max(o, axis=1, keepdims=True)
    e = jnp.exp(o - m)
    s = jnp.sum(e, axis=1, keepdims=True)
    out_ref[...] = o - m - jnp.log(s)


def kernel(x, edge_index, W_conv, b_conv, W_fc, b_fc):
    n = x.shape[0]
    e = edge_index.shape[1]
    d_in = x.shape[1]

    n_pad = ((n + 1 + NS * 8 - 1) // (NS * 8)) * (NS * 8)
    cpw = -(-e // (NW * CHUNK))
    cpw = ((cpw + 15) // 16) * 16

    src = edge_index[0].astype(jnp.int32)
    dst = edge_index[1].astype(jnp.int32)
    e_pad = NW * CHUNK * cpw
    spread = jnp.arange(e_pad - e, dtype=jnp.int32) % (n_pad - n)
    fill = n + spread
    src2d = jnp.concatenate([src, fill]).reshape(e_pad // CHUNK, CHUNK)
    dst2d = jnp.concatenate([dst, fill]).reshape(e_pad // CHUNK, CHUNK)
    ones = jnp.ones((n_pad,), jnp.float32)

    degp, psrc, pdst, cnts = _partition_call(src2d, dst2d, ones, n_pad, cpw)
    degt = jnp.swapaxes(degp, 0, 1)

    blk = 8192
    grid = -(-n // blk)

    g = pl.pallas_call(
        _dense1_body,
        grid=(grid,),
        in_specs=[
            pl.BlockSpec((blk, d_in), lambda i: (i, 0)),
            pl.BlockSpec((blk, 2), lambda i: (i, 0)),
            pl.BlockSpec((d_in, D_HID), lambda i: (0, 0)),
        ],
        out_specs=pl.BlockSpec((blk, D_HID), lambda i: (i, 0)),
        out_shape=jax.ShapeDtypeStruct((n_pad, D_HID), jnp.float32),
    )(x, degt, W_conv)

    acc = _edge_scatter_call(psrc, pdst, cnts, g, n_pad, cpw)

    out = pl.pallas_call(
        _dense2_body,
        grid=(grid,),
        in_specs=[
            pl.BlockSpec((blk, D_HID), lambda i: (i, 0)),
            pl.BlockSpec((blk, 2), lambda i: (i, 0)),
            pl.BlockSpec((1, D_HID), lambda i: (0, 0)),
            pl.BlockSpec((D_HID, D_OUT), lambda i: (0, 0)),
            pl.BlockSpec((1, D_OUT), lambda i: (0, 0)),
        ],
        out_specs=pl.BlockSpec((blk, D_OUT), lambda i: (i, 0)),
        out_shape=jax.ShapeDtypeStruct((n, D_OUT), jnp.float32),
    )(acc, degt, b_conv.reshape(1, D_HID), W_fc, b_fc.reshape(1, D_OUT))

    return out
